# baseline (speedup 1.0000x reference)
import jax
import jax.numpy as jnp
from jax import lax
from jax.experimental import pallas as pl
from jax.experimental.pallas import tpu as pltpu
from jax.experimental.pallas import tpu_sc as plsc

_KC = 1389.35457644382
_ADIV = 1.0 / (0.8854 * 0.5291772105638411)
_CUTOFF = 5.0
_CUTOFF_SWITCH = _CUTOFF - 1.0
_APOW = 0.23
_C1, _C2, _C3, _C4 = 0.1818, 0.5099, 0.2802, 0.02817
_A1, _A2, _A3, _A4 = 3.2, 0.9423, 0.4028, 0.2016
_SA, _SB, _SC = 6.0, 15.0, 10.0

_NC, _NS, _L = 2, 16, 16
_NW = _NC * _NS
_NB = 64
_ZTAB = 128


def _pick_chunk(n, cap):
    best = _L
    c = _L
    lim = min(n, cap)
    while c <= lim:
        if n % c == 0:
            best = c
        c += _L
    return best


def kernel(R1_esp, senders_esp, receivers_esp, Z, batch_index_esp):
    E = senders_esp.shape[0]
    N = Z.shape[0]
    assert E % _NW == 0 and N % _L == 0
    epw = E // _NW
    ch = _pick_chunk(epw, 10000)
    n_ech = epw // ch
    nvec = ch // _L

    za_tab = jnp.arange(_ZTAB, dtype=jnp.float32) ** _APOW
    R_flat = R1_esp.reshape(-1)
    s_i32 = senders_esp.astype(jnp.int32)
    r_i32 = receivers_esp.astype(jnp.int32)
    z_i32 = Z.astype(jnp.int32)
    b_i32 = batch_index_esp.astype(jnp.int32)

    mesh = plsc.VectorSubcoreMesh(
        core_axis_name="c", subcore_axis_name="s",
        num_cores=_NC, num_subcores=_NS)

    def body(R_hbm, s_hbm, r_hbm, b_hbm, z_hbm, za_hbm, out_hbm,
             zpack_v, za_v, s_v, r_v, R_v, b_v, acc_v):
        wid = lax.axis_index("s") * _NC + lax.axis_index("c")
        pltpu.sync_copy(za_hbm, za_v)
        pltpu.sync_copy(z_hbm, zpack_v)

        @plsc.parallel_loop(0, N // _L, unroll=8)
        def pack_vec(i):
            off = i * _L
            zz = zpack_v[pl.ds(off, _L)]
            za = plsc.load_gather(za_v, [zz])
            w = jnp.bitwise_or(
                jnp.bitwise_and(plsc.bitcast(za, jnp.int32), -128), zz)
            zpack_v[pl.ds(off, _L)] = w

        zero = jnp.zeros((_L,), jnp.float32)
        for k in range(_NB // _L):
            acc_v[pl.ds(k * _L, _L)] = zero

        ebase = wid * epw

        def chunk(ci, carry):
            base = ebase + ci * ch
            pltpu.sync_copy(s_hbm.at[pl.ds(base, ch)], s_v)
            pltpu.sync_copy(r_hbm.at[pl.ds(base, ch)], r_v)
            pltpu.sync_copy(R_hbm.at[pl.ds(base, ch)], R_v)
            pltpu.sync_copy(b_hbm.at[pl.ds(base, ch)], b_v)

            @plsc.parallel_loop(0, nvec, unroll=8)
            def vec(vi):
                off = vi * _L
                s16 = s_v[pl.ds(off, _L)]
                r16 = r_v[pl.ds(off, _L)]
                R16 = R_v[pl.ds(off, _L)]
                b16 = b_v[pl.ds(off, _L)]
                ws = plsc.load_gather(zpack_v, [s16])
                wr = plsc.load_gather(zpack_v, [r16])
                zs = jnp.bitwise_and(ws, 127).astype(jnp.float32)
                zr = jnp.bitwise_and(wr, 127).astype(jnp.float32)
                zas = plsc.bitcast(jnp.bitwise_and(ws, -128), jnp.float32)
                zar = plsc.bitcast(jnp.bitwise_and(wr, -128), jnp.float32)
                aR = (zas + zar) * _ADIV * R16
                phi = (_C1 * jnp.exp(aR * (-_A1)) + _C2 * jnp.exp(aR * (-_A2))
                       + _C3 * jnp.exp(aR * (-_A3)) + _C4 * jnp.exp(aR * (-_A4)))
                X = R16 - _CUTOFF_SWITCH
                X2 = X * X
                X3 = X2 * X
                X4 = X2 * X2
                X5 = X4 * X
                sw = jnp.clip(1.0 - _SA * X5 + _SB * X4 - _SC * X3, 0.0, 1.0)
                term = _KC * sw * phi * (zs * zr) / R16
                term = jnp.where(R16 < _CUTOFF, term, 0.0)
                plsc.addupdate_scatter(acc_v, [b16], term)
            return carry
        lax.fori_loop(0, n_ech, chunk, 0)
        pltpu.sync_copy(acc_v, out_hbm.at[wid])

    run = pl.kernel(
        body,
        out_type=jax.ShapeDtypeStruct((_NW, _NB), jnp.float32),
        mesh=mesh,
        compiler_params=pltpu.CompilerParams(needs_layout_passes=False),
        scratch_types=[
            pltpu.VMEM((N,), jnp.int32),
            pltpu.VMEM((_ZTAB,), jnp.float32),
            pltpu.VMEM((ch,), jnp.int32),
            pltpu.VMEM((ch,), jnp.int32),
            pltpu.VMEM((ch,), jnp.float32),
            pltpu.VMEM((ch,), jnp.int32),
            pltpu.VMEM((_NB,), jnp.float32),
        ],
    )
    partials = run(R_flat, s_i32, r_i32, b_i32, z_i32, za_tab)
    return partials.sum(axis=0, dtype=jnp.float32).reshape(_NB, 1)


# speedup vs baseline: 2.3061x; 2.3061x over previous
import jax
import jax.numpy as jnp
from jax import lax
from jax.experimental import pallas as pl
from jax.experimental.pallas import tpu as pltpu
from jax.experimental.pallas import tpu_sc as plsc

_KC = 1389.35457644382
_ADIV = 1.0 / (0.8854 * 0.5291772105638411)
_CUTOFF = 5.0
_CUTOFF_SWITCH = _CUTOFF - 1.0
_APOW = 0.23
_C1, _C2, _C3, _C4 = 0.1818, 0.5099, 0.2802, 0.02817
_A1, _A2, _A3, _A4 = 3.2, 0.9423, 0.4028, 0.2016
_SA, _SB, _SC = 6.0, 15.0, 10.0

_NC, _NS, _L = 2, 16, 16
_NW = _NC * _NS
_NB = 64
_ZTAB = 128


def _pick_chunk(n, cap):
    best = _L
    c = _L
    lim = min(n, cap)
    while c <= lim:
        if n % c == 0:
            best = c
        c += _L
    return best


def _pick_chunk_even(n, cap):
    best = _L
    c = _L
    lim = min(n, cap)
    while c <= lim:
        if n % (2 * c) == 0:
            best = c
        c += _L
    return best


def kernel(R1_esp, senders_esp, receivers_esp, Z, batch_index_esp):
    E = senders_esp.shape[0]
    N = Z.shape[0]
    assert E % _NW == 0 and N % _L == 0
    epw = E // _NW
    ch = _pick_chunk_even(epw, 4000)
    n_ech = epw // ch
    nvec = ch // _L

    za_tab = jnp.arange(_ZTAB, dtype=jnp.float32) ** _APOW
    R_flat = R1_esp.reshape(-1)
    s_i32 = senders_esp.astype(jnp.int32)
    r_i32 = receivers_esp.astype(jnp.int32)
    z_i32 = Z.astype(jnp.int32)
    b_i32 = batch_index_esp.astype(jnp.int32)

    mesh = plsc.VectorSubcoreMesh(
        core_axis_name="c", subcore_axis_name="s",
        num_cores=_NC, num_subcores=_NS)

    def body(R_hbm, s_hbm, r_hbm, b_hbm, z_hbm, za_hbm, out_hbm,
             zpack_v, za_v, sA, rA, RA, bA, sB, rB, RB, bB,
             acc_v, semA, semB):
        wid = lax.axis_index("s") * _NC + lax.axis_index("c")
        pltpu.sync_copy(za_hbm, za_v)
        pltpu.sync_copy(z_hbm, zpack_v)

        @plsc.parallel_loop(0, N // _L, unroll=8)
        def pack_vec(i):
            off = i * _L
            zz = zpack_v[pl.ds(off, _L)]
            za = plsc.load_gather(za_v, [zz])
            w = jnp.bitwise_or(
                jnp.bitwise_and(plsc.bitcast(za, jnp.int32), -128), zz)
            zpack_v[pl.ds(off, _L)] = w

        zero = jnp.zeros((_L,), jnp.float32)
        for k in range(_NB // _L):
            acc_v[pl.ds(k * _L, _L)] = zero

        ebase = wid * epw
        hbm_srcs = (s_hbm, r_hbm, R_hbm, b_hbm)
        bufsA = (sA, rA, RA, bA)
        bufsB = (sB, rB, RB, bB)

        def start_fetch(bufs, sem, base):
            for hb, dst in zip(hbm_srcs, bufs):
                pltpu.async_copy(hb.at[pl.ds(base, ch)], dst, sem)

        def drain(bufs, sem):
            for hb, dst in zip(hbm_srcs, bufs):
                pltpu.make_async_copy(hb.at[pl.ds(0, ch)], dst, sem).wait()

        def compute(bufs):
            s_v, r_v, R_v, b_v = bufs

            @plsc.parallel_loop(0, nvec, unroll=4)
            def vec(vi):
                off = vi * _L
                s16 = s_v[pl.ds(off, _L)]
                r16 = r_v[pl.ds(off, _L)]
                R16 = R_v[pl.ds(off, _L)]
                b16 = b_v[pl.ds(off, _L)]
                ws = plsc.load_gather(zpack_v, [s16])
                wr = plsc.load_gather(zpack_v, [r16])
                zs = jnp.bitwise_and(ws, 127).astype(jnp.float32)
                zr = jnp.bitwise_and(wr, 127).astype(jnp.float32)
                zas = plsc.bitcast(jnp.bitwise_and(ws, -128), jnp.float32)
                zar = plsc.bitcast(jnp.bitwise_and(wr, -128), jnp.float32)
                aR = (zas + zar) * _ADIV * R16
                phi = (_C1 * jnp.exp(aR * (-_A1)) + _C2 * jnp.exp(aR * (-_A2))
                       + _C3 * jnp.exp(aR * (-_A3)) + _C4 * jnp.exp(aR * (-_A4)))
                X = R16 - _CUTOFF_SWITCH
                X2 = X * X
                X3 = X2 * X
                X4 = X2 * X2
                X5 = X4 * X
                sw = jnp.clip(1.0 - _SA * X5 + _SB * X4 - _SC * X3, 0.0, 1.0)
                term = _KC * sw * phi * (zs * zr) / R16
                term = jnp.where(R16 < _CUTOFF, term, 0.0)
                plsc.addupdate_scatter(acc_v, [b16], term)

        start_fetch(bufsA, semA, ebase)

        def pair(p, carry):
            start_fetch(bufsB, semB, ebase + (2 * p + 1) * ch)
            drain(bufsA, semA)
            compute(bufsA)
            nxt = 2 * p + 2

            @pl.when(nxt < n_ech)
            def _():
                start_fetch(bufsA, semA, ebase + nxt * ch)
            drain(bufsB, semB)
            compute(bufsB)
            return carry
        lax.fori_loop(0, n_ech // 2, pair, 0)
        pltpu.sync_copy(acc_v, out_hbm.at[wid])

    run = pl.kernel(
        body,
        out_type=jax.ShapeDtypeStruct((_NW, _NB), jnp.float32),
        mesh=mesh,
        compiler_params=pltpu.CompilerParams(needs_layout_passes=False),
        scratch_types=[
            pltpu.VMEM((N,), jnp.int32),
            pltpu.VMEM((_ZTAB,), jnp.float32),
            pltpu.VMEM((ch,), jnp.int32),
            pltpu.VMEM((ch,), jnp.int32),
            pltpu.VMEM((ch,), jnp.float32),
            pltpu.VMEM((ch,), jnp.int32),
            pltpu.VMEM((ch,), jnp.int32),
            pltpu.VMEM((ch,), jnp.int32),
            pltpu.VMEM((ch,), jnp.float32),
            pltpu.VMEM((ch,), jnp.int32),
            pltpu.VMEM((_NB,), jnp.float32),
            pltpu.SemaphoreType.DMA,
            pltpu.SemaphoreType.DMA,
        ],
    )
    partials = run(R_flat, s_i32, r_i32, b_i32, z_i32, za_tab)
    return partials.sum(axis=0, dtype=jnp.float32).reshape(_NB, 1)
